# baseline (speedup 1.0000x reference)
import jax
import jax.numpy as jnp
import numpy as np
from jax import lax
from jax.experimental import pallas as pl
from jax.experimental.pallas import tpu as pltpu
from jax.experimental.pallas import tpu_sc as plsc

N_POINTS = 262144
NUM_CORES = 2
NUM_SUBCORES = 16
NUM_TILES = NUM_CORES * NUM_SUBCORES
PER_TILE = N_POINTS // NUM_TILES
LANES = 16
VEC_ITERS = PER_TILE // LANES

SIZES = (np.float32(0.1), np.float32(0.2), np.float32(0.4))
GRIDS = (1024, 512, 256)
BASES = (0, 1024 * 1024, 1024 * 1024 + 512 * 512)
C = 1024 * 1024 + 512 * 512 + 256 * 256
SHIFT = np.float32(51.2)

CHUNK = 128
N_ROWS = 3 * PER_TILE // CHUNK
ROWS_PER_RES = N_ROWS // 3
ROWS_PER_STREAM = 8
N_STREAMS = N_ROWS // ROWS_PER_STREAM
ZB = 2048
ZERO_PER_SUBCORE = C // NUM_SUBCORES
N_ZERO_COPIES = ZERO_PER_SUBCORE // ZB


def _scatter_body(xs_hbm, ys_hbm, out_hbm, xv, yv, idxb, ones, zb, grid_sh,
                  sem_ld, sem_sc, sem_z):
    cid = lax.axis_index("c")
    sid = lax.axis_index("s")
    wid = cid * NUM_SUBCORES + sid

    base = wid * PER_TILE
    d_x = pltpu.async_copy(xs_hbm.at[pl.ds(base, PER_TILE)], xv, sem_ld)
    d_y = pltpu.async_copy(ys_hbm.at[pl.ds(base, PER_TILE)], yv, sem_ld)

    def _fill_zb(i, _):
        zb[pl.ds(i * LANES, LANES)] = jnp.zeros((LANES,), jnp.float32)
        return 0

    lax.fori_loop(0, ZB // LANES, _fill_zb, 0)

    def _fill_ones(i, _):
        ones[i // 8, pl.ds((i % 8) * LANES, LANES)] = jnp.ones(
            (LANES,), jnp.float32)
        return 0

    lax.fori_loop(0, ROWS_PER_STREAM * CHUNK // LANES, _fill_ones, 0)
    ones_r = ones.at[0]

    zbase = sid * ZERO_PER_SUBCORE

    def _zero_fire(j, _):
        pltpu.async_copy(zb, grid_sh.at[pl.ds(zbase + j * ZB, ZB)], sem_z)
        return 0

    lax.fori_loop(0, N_ZERO_COPIES, _zero_fire, 0)

    d_x.wait()
    d_y.wait()

    def _quant(i, _):
        x = xv[pl.ds(i * LANES, LANES)]
        y = yv[pl.ds(i * LANES, LANES)]
        sx = x + SHIFT
        sy = y + SHIFT
        row = i // 8
        col = (i % 8) * LANES
        cx = (sx / SIZES[0]).astype(jnp.int32)
        cy = (sy / SIZES[0]).astype(jnp.int32)
        cx = jnp.minimum(jnp.maximum(cx, 0), GRIDS[0] - 1)
        cy = jnp.minimum(jnp.maximum(cy, 0), GRIDS[0] - 1)
        idxb[row, pl.ds(col, LANES)] = cy * 1024 + cx
        idxb[ROWS_PER_RES + row, pl.ds(col, LANES)] = (
            (cy >> 1) * 512 + (cx >> 1) + BASES[1])
        idxb[2 * ROWS_PER_RES + row, pl.ds(col, LANES)] = (
            (cy >> 2) * 256 + (cx >> 2) + BASES[2])
        return 0

    with jax.named_scope("ph_quant"):
        lax.fori_loop(0, VEC_ITERS, _quant, 0)

    def _zero_drain(j, _):
        pltpu.make_async_copy(
            zb, grid_sh.at[pl.ds(zbase + j * ZB, ZB)], sem_z).wait()
        return 0

    with jax.named_scope("ph_zdrain"):
        lax.fori_loop(0, N_ZERO_COPIES, _zero_drain, 0)
        plsc.subcore_barrier()

    def _scatter_fire(j, _):
        pltpu.async_copy(ones_r, grid_sh.at[idxb.at[j]], sem_sc)
        return 0

    with jax.named_scope("ph_sfire"):
        lax.fori_loop(0, N_ROWS, _scatter_fire, 0)

    def _scatter_drain(j, _):
        pltpu.make_async_copy(ones_r, grid_sh.at[idxb.at[j]], sem_sc).wait()
        return 0

    with jax.named_scope("ph_sdrain"):
        lax.fori_loop(0, N_ROWS, _scatter_drain, 0)
        plsc.subcore_barrier()

    with jax.named_scope("ph_wb"):
        pltpu.sync_copy(grid_sh.at[pl.ds(zbase, ZERO_PER_SUBCORE)],
                        out_hbm.at[pl.ds(cid * C + zbase, ZERO_PER_SUBCORE)])


_scatter_call = pl.kernel(
    _scatter_body,
    out_type=jax.ShapeDtypeStruct((NUM_CORES * C,), jnp.float32),
    mesh=plsc.VectorSubcoreMesh(core_axis_name="c", subcore_axis_name="s"),
    scratch_types=[
        pltpu.VMEM((PER_TILE,), jnp.float32),
        pltpu.VMEM((PER_TILE,), jnp.float32),
        pltpu.VMEM((N_ROWS, CHUNK), jnp.int32),
        pltpu.VMEM((ROWS_PER_STREAM, CHUNK), jnp.float32),
        pltpu.VMEM((ZB,), jnp.float32),
        pltpu.VMEM_SHARED((C,), jnp.float32),
        pltpu.SemaphoreType.DMA,
        pltpu.SemaphoreType.DMA,
        pltpu.SemaphoreType.DMA,
    ],
)

S0 = 32 * 1024
S1 = 32 * 512
S2 = 32 * 256


def _reduce_body(a0, a1, b0, b1, c0, c1, o0_ref, o1_ref, o2_ref):
    b = pl.program_id(0)

    def cnt(r0, r1):
        n = r0.shape[0]
        a = r0[...].reshape(n // 1024, 1024)
        b2 = r1[...].reshape(n // 1024, 1024)
        occ = ((a + b2) > 0.0).astype(jnp.float32)
        return jnp.sum(occ).astype(jnp.int32)

    def put(ref, lane, val):
        li = lax.broadcasted_iota(jnp.int32, ref.shape, 2)
        ref[...] = jnp.where(li == lane, val, ref[...])

    put(o0_ref, b, cnt(a0, a1))
    put(o1_ref, b % 16, cnt(b0, b1))
    put(o2_ref, b % 8, cnt(c0, c1))


_reduce_call = pl.pallas_call(
    _reduce_body,
    grid=(32,),
    in_specs=[
        pl.BlockSpec((S0,), lambda b: (b,)),
        pl.BlockSpec((S0,), lambda b: (C // S0 + b,)),
        pl.BlockSpec((S1,), lambda b: (BASES[1] // S1 + b % 16,)),
        pl.BlockSpec((S1,), lambda b: ((C + BASES[1]) // S1 + b % 16,)),
        pl.BlockSpec((S2,), lambda b: (BASES[2] // S2 + b % 8,)),
        pl.BlockSpec((S2,), lambda b: ((C + BASES[2]) // S2 + b % 8,)),
    ],
    out_specs=[
        pl.BlockSpec((1, 1, 32), lambda b: (0, 0, 0)),
        pl.BlockSpec((1, 1, 16), lambda b: (0, 0, 0)),
        pl.BlockSpec((1, 1, 8), lambda b: (0, 0, 0)),
    ],
    out_shape=[
        jax.ShapeDtypeStruct((1, 1, 32), jnp.int32),
        jax.ShapeDtypeStruct((1, 1, 16), jnp.int32),
        jax.ShapeDtypeStruct((1, 1, 8), jnp.int32),
    ],
)


def kernel(points_xy):
    grids = _scatter_call(points_xy[:, 0], points_xy[:, 1])
    o0, o1, o2 = _reduce_call(grids, grids, grids, grids, grids, grids)
    return (o0.reshape(1, 32), o1.reshape(1, 16), o2.reshape(1, 8))


# speedup vs baseline: 1.0614x; 1.0614x over previous
import jax
import jax.numpy as jnp
import numpy as np
from jax import lax
from jax.experimental import pallas as pl
from jax.experimental.pallas import tpu as pltpu
from jax.experimental.pallas import tpu_sc as plsc

N_POINTS = 262144
NUM_CORES = 2
NUM_SUBCORES = 16
NUM_TILES = NUM_CORES * NUM_SUBCORES
PER_TILE = N_POINTS // NUM_TILES
LANES = 16
VEC_ITERS = PER_TILE // LANES

SIZES = (np.float32(0.1), np.float32(0.2), np.float32(0.4))
GRIDS = (1024, 512, 256)
BASES = (0, 1024 * 1024, 1024 * 1024 + 512 * 512)
C = 1024 * 1024 + 512 * 512 + 256 * 256
SHIFT = np.float32(51.2)

CHUNK = 128
N_ROWS = 3 * PER_TILE // CHUNK
ROWS_PER_RES = N_ROWS // 3
ROWS_PER_STREAM = 8
N_STREAMS = N_ROWS // ROWS_PER_STREAM
ZB = 2048
ZERO_PER_SUBCORE = C // NUM_SUBCORES
N_ZERO_COPIES = ZERO_PER_SUBCORE // ZB


def _scatter_body(xs_hbm, ys_hbm, out_hbm, xv, yv, idxb, ones, zb, grid_sh,
                  sem_ld, sem_sc, sem_z):
    cid = lax.axis_index("c")
    sid = lax.axis_index("s")
    wid = cid * NUM_SUBCORES + sid

    base = wid * PER_TILE
    d_x = pltpu.async_copy(xs_hbm.at[pl.ds(base, PER_TILE)], xv, sem_ld)
    d_y = pltpu.async_copy(ys_hbm.at[pl.ds(base, PER_TILE)], yv, sem_ld)

    @plsc.parallel_loop(0, ZB // LANES, unroll=8)
    def _fill_zb(i):
        zb[pl.ds(i * LANES, LANES)] = jnp.zeros((LANES,), jnp.float32)

    @plsc.parallel_loop(0, ROWS_PER_STREAM * CHUNK // LANES, unroll=8)
    def _fill_ones(i):
        ones[i // 8, pl.ds((i % 8) * LANES, LANES)] = jnp.ones(
            (LANES,), jnp.float32)

    ones_r = ones.at[0]

    zbase = sid * ZERO_PER_SUBCORE

    def _zero_fire(j, _):
        pltpu.async_copy(zb, grid_sh.at[pl.ds(zbase + j * ZB, ZB)], sem_z)
        return 0

    lax.fori_loop(0, N_ZERO_COPIES, _zero_fire, 0)

    d_x.wait()
    d_y.wait()

    def _quant(i):
        x = xv[pl.ds(i * LANES, LANES)]
        y = yv[pl.ds(i * LANES, LANES)]
        sx = x + SHIFT
        sy = y + SHIFT
        row = i // 8
        col = (i % 8) * LANES
        cx = (sx / SIZES[0]).astype(jnp.int32)
        cy = (sy / SIZES[0]).astype(jnp.int32)
        cx = jnp.minimum(jnp.maximum(cx, 0), GRIDS[0] - 1)
        cy = jnp.minimum(jnp.maximum(cy, 0), GRIDS[0] - 1)
        idxb[row, pl.ds(col, LANES)] = cy * 1024 + cx
        idxb[ROWS_PER_RES + row, pl.ds(col, LANES)] = (
            (cy >> 1) * 512 + (cx >> 1) + BASES[1])
        idxb[2 * ROWS_PER_RES + row, pl.ds(col, LANES)] = (
            (cy >> 2) * 256 + (cx >> 2) + BASES[2])

    with jax.named_scope("ph_quant"):
        plsc.parallel_loop(0, VEC_ITERS, unroll=4)(_quant)

    def _zero_drain(j, _):
        pltpu.make_async_copy(
            zb, grid_sh.at[pl.ds(zbase + j * ZB, ZB)], sem_z).wait()
        return 0

    with jax.named_scope("ph_zdrain"):
        lax.fori_loop(0, N_ZERO_COPIES, _zero_drain, 0)
        plsc.subcore_barrier()

    def _scatter_fire(j, _):
        pltpu.async_copy(ones_r, grid_sh.at[idxb.at[j]], sem_sc)
        return 0

    with jax.named_scope("ph_sfire"):
        lax.fori_loop(0, N_ROWS, _scatter_fire, 0)

    def _scatter_drain(j, _):
        pltpu.make_async_copy(ones_r, grid_sh.at[idxb.at[j]], sem_sc).wait()
        return 0

    with jax.named_scope("ph_sdrain"):
        lax.fori_loop(0, N_ROWS, _scatter_drain, 0)
        plsc.subcore_barrier()

    with jax.named_scope("ph_wb"):
        pltpu.sync_copy(grid_sh.at[pl.ds(zbase, ZERO_PER_SUBCORE)],
                        out_hbm.at[pl.ds(cid * C + zbase, ZERO_PER_SUBCORE)])


_scatter_call = pl.kernel(
    _scatter_body,
    out_type=jax.ShapeDtypeStruct((NUM_CORES * C,), jnp.float32),
    mesh=plsc.VectorSubcoreMesh(core_axis_name="c", subcore_axis_name="s"),
    scratch_types=[
        pltpu.VMEM((PER_TILE,), jnp.float32),
        pltpu.VMEM((PER_TILE,), jnp.float32),
        pltpu.VMEM((N_ROWS, CHUNK), jnp.int32),
        pltpu.VMEM((ROWS_PER_STREAM, CHUNK), jnp.float32),
        pltpu.VMEM((ZB,), jnp.float32),
        pltpu.VMEM_SHARED((C,), jnp.float32),
        pltpu.SemaphoreType.DMA,
        pltpu.SemaphoreType.DMA,
        pltpu.SemaphoreType.DMA,
    ],
)

S0 = 32 * 1024
S1 = 32 * 512
S2 = 32 * 256


def _reduce_body(a0, a1, b0, b1, c0, c1, o0_ref, o1_ref, o2_ref):
    b = pl.program_id(0)

    def cnt(r0, r1):
        n = r0.shape[0]
        a = r0[...].reshape(n // 1024, 1024)
        b2 = r1[...].reshape(n // 1024, 1024)
        occ = ((a + b2) > 0.0).astype(jnp.float32)
        return jnp.sum(occ).astype(jnp.int32)

    def put(ref, lane, val):
        li = lax.broadcasted_iota(jnp.int32, ref.shape, 2)
        ref[...] = jnp.where(li == lane, val, ref[...])

    put(o0_ref, b, cnt(a0, a1))
    put(o1_ref, b % 16, cnt(b0, b1))
    put(o2_ref, b % 8, cnt(c0, c1))


_reduce_call = pl.pallas_call(
    _reduce_body,
    grid=(32,),
    in_specs=[
        pl.BlockSpec((S0,), lambda b: (b,)),
        pl.BlockSpec((S0,), lambda b: (C // S0 + b,)),
        pl.BlockSpec((S1,), lambda b: (BASES[1] // S1 + b % 16,)),
        pl.BlockSpec((S1,), lambda b: ((C + BASES[1]) // S1 + b % 16,)),
        pl.BlockSpec((S2,), lambda b: (BASES[2] // S2 + b % 8,)),
        pl.BlockSpec((S2,), lambda b: ((C + BASES[2]) // S2 + b % 8,)),
    ],
    out_specs=[
        pl.BlockSpec((1, 1, 32), lambda b: (0, 0, 0)),
        pl.BlockSpec((1, 1, 16), lambda b: (0, 0, 0)),
        pl.BlockSpec((1, 1, 8), lambda b: (0, 0, 0)),
    ],
    out_shape=[
        jax.ShapeDtypeStruct((1, 1, 32), jnp.int32),
        jax.ShapeDtypeStruct((1, 1, 16), jnp.int32),
        jax.ShapeDtypeStruct((1, 1, 8), jnp.int32),
    ],
)


def kernel(points_xy):
    grids = _scatter_call(points_xy[:, 0], points_xy[:, 1])
    o0, o1, o2 = _reduce_call(grids, grids, grids, grids, grids, grids)
    return (o0.reshape(1, 32), o1.reshape(1, 16), o2.reshape(1, 8))


# speedup vs baseline: 1.0693x; 1.0074x over previous
import jax
import jax.numpy as jnp
import numpy as np
from jax import lax
from jax.experimental import pallas as pl
from jax.experimental.pallas import tpu as pltpu
from jax.experimental.pallas import tpu_sc as plsc

N_POINTS = 262144
NUM_CORES = 2
NUM_SUBCORES = 16
NUM_TILES = NUM_CORES * NUM_SUBCORES
PER_TILE = N_POINTS // NUM_TILES
LANES = 16
VEC_ITERS = PER_TILE // LANES

SIZES = (np.float32(0.1), np.float32(0.2), np.float32(0.4))
GRIDS = (1024, 512, 256)
BASES = (0, 1024 * 1024, 1024 * 1024 + 512 * 512)
C = 1024 * 1024 + 512 * 512 + 256 * 256
SHIFT = np.float32(51.2)

CHUNK = 128
N_ROWS = 3 * PER_TILE // CHUNK
ROWS_PER_RES = N_ROWS // 3
ROWS_PER_STREAM = 8
N_STREAMS = N_ROWS // ROWS_PER_STREAM
ZB = 2048
ZERO_PER_SUBCORE = C // NUM_SUBCORES
N_ZERO_COPIES = ZERO_PER_SUBCORE // ZB


def _scatter_body(xs_hbm, ys_hbm, out_hbm, xv, yv, idxb, ones, zb, grid_sh,
                  sem_ld, sem_sc, sem_z):
    cid = lax.axis_index("c")
    sid = lax.axis_index("s")
    wid = cid * NUM_SUBCORES + sid

    base = wid * PER_TILE
    d_x = pltpu.async_copy(xs_hbm.at[pl.ds(base, PER_TILE)], xv, sem_ld)
    d_y = pltpu.async_copy(ys_hbm.at[pl.ds(base, PER_TILE)], yv, sem_ld)

    @plsc.parallel_loop(0, ZB // LANES, unroll=8)
    def _fill_zb(i):
        zb[pl.ds(i * LANES, LANES)] = jnp.zeros((LANES,), jnp.float32)

    @plsc.parallel_loop(0, ROWS_PER_STREAM * CHUNK // LANES, unroll=8)
    def _fill_ones(i):
        ones[i // 8, pl.ds((i % 8) * LANES, LANES)] = jnp.ones(
            (LANES,), jnp.float32)

    ones_r = ones.at[0]

    zbase = sid * ZERO_PER_SUBCORE

    @plsc.parallel_loop(0, N_ZERO_COPIES, unroll=4)
    def _zero_fire(j):
        pltpu.async_copy(zb, grid_sh.at[pl.ds(zbase + j * ZB, ZB)], sem_z)

    d_x.wait()
    d_y.wait()

    def _quant(i):
        x = xv[pl.ds(i * LANES, LANES)]
        y = yv[pl.ds(i * LANES, LANES)]
        sx = x + SHIFT
        sy = y + SHIFT
        row = i // 8
        col = (i % 8) * LANES
        cx = (sx / SIZES[0]).astype(jnp.int32)
        cy = (sy / SIZES[0]).astype(jnp.int32)
        cx = jnp.minimum(jnp.maximum(cx, 0), GRIDS[0] - 1)
        cy = jnp.minimum(jnp.maximum(cy, 0), GRIDS[0] - 1)
        idxb[row, pl.ds(col, LANES)] = cy * 1024 + cx
        idxb[ROWS_PER_RES + row, pl.ds(col, LANES)] = (
            (cy >> 1) * 512 + (cx >> 1) + BASES[1])
        idxb[2 * ROWS_PER_RES + row, pl.ds(col, LANES)] = (
            (cy >> 2) * 256 + (cx >> 2) + BASES[2])

    with jax.named_scope("ph_quant"):
        plsc.parallel_loop(0, VEC_ITERS, unroll=4)(_quant)

    def _zero_drain(j, _):
        pltpu.make_async_copy(
            zb, grid_sh.at[pl.ds(zbase + j * ZB, ZB)], sem_z).wait()
        return 0

    with jax.named_scope("ph_zdrain"):
        lax.fori_loop(0, N_ZERO_COPIES, _zero_drain, 0)
        plsc.subcore_barrier()

    def _scatter_fire(j):
        pltpu.async_copy(ones_r, grid_sh.at[idxb.at[j]], sem_sc)

    with jax.named_scope("ph_sfire"):
        plsc.parallel_loop(0, N_ROWS, unroll=4)(_scatter_fire)

    def _scatter_drain(j, _):
        pltpu.make_async_copy(ones_r, grid_sh.at[idxb.at[j]], sem_sc).wait()
        return 0

    with jax.named_scope("ph_sdrain"):
        lax.fori_loop(0, N_ROWS, _scatter_drain, 0)
        plsc.subcore_barrier()

    with jax.named_scope("ph_wb"):
        pltpu.sync_copy(grid_sh.at[pl.ds(zbase, ZERO_PER_SUBCORE)],
                        out_hbm.at[pl.ds(cid * C + zbase, ZERO_PER_SUBCORE)])


_scatter_call = pl.kernel(
    _scatter_body,
    out_type=jax.ShapeDtypeStruct((NUM_CORES * C,), jnp.float32),
    mesh=plsc.VectorSubcoreMesh(core_axis_name="c", subcore_axis_name="s"),
    scratch_types=[
        pltpu.VMEM((PER_TILE,), jnp.float32),
        pltpu.VMEM((PER_TILE,), jnp.float32),
        pltpu.VMEM((N_ROWS, CHUNK), jnp.int32),
        pltpu.VMEM((ROWS_PER_STREAM, CHUNK), jnp.float32),
        pltpu.VMEM((ZB,), jnp.float32),
        pltpu.VMEM_SHARED((C,), jnp.float32),
        pltpu.SemaphoreType.DMA,
        pltpu.SemaphoreType.DMA,
        pltpu.SemaphoreType.DMA,
    ],
)

S0 = 32 * 1024
S1 = 32 * 512
S2 = 32 * 256


def _reduce_body(a0, a1, b0, b1, c0, c1, o0_ref, o1_ref, o2_ref):
    b = pl.program_id(0)

    def cnt(r0, r1):
        n = r0.shape[0]
        a = r0[...].reshape(n // 1024, 1024)
        b2 = r1[...].reshape(n // 1024, 1024)
        occ = ((a + b2) > 0.0).astype(jnp.float32)
        return jnp.sum(occ).astype(jnp.int32)

    def put(ref, lane, val):
        li = lax.broadcasted_iota(jnp.int32, ref.shape, 2)
        ref[...] = jnp.where(li == lane, val, ref[...])

    put(o0_ref, b, cnt(a0, a1))
    put(o1_ref, b % 16, cnt(b0, b1))
    put(o2_ref, b % 8, cnt(c0, c1))


_reduce_call = pl.pallas_call(
    _reduce_body,
    grid=(32,),
    in_specs=[
        pl.BlockSpec((S0,), lambda b: (b,)),
        pl.BlockSpec((S0,), lambda b: (C // S0 + b,)),
        pl.BlockSpec((S1,), lambda b: (BASES[1] // S1 + b % 16,)),
        pl.BlockSpec((S1,), lambda b: ((C + BASES[1]) // S1 + b % 16,)),
        pl.BlockSpec((S2,), lambda b: (BASES[2] // S2 + b % 8,)),
        pl.BlockSpec((S2,), lambda b: ((C + BASES[2]) // S2 + b % 8,)),
    ],
    out_specs=[
        pl.BlockSpec((1, 1, 32), lambda b: (0, 0, 0)),
        pl.BlockSpec((1, 1, 16), lambda b: (0, 0, 0)),
        pl.BlockSpec((1, 1, 8), lambda b: (0, 0, 0)),
    ],
    out_shape=[
        jax.ShapeDtypeStruct((1, 1, 32), jnp.int32),
        jax.ShapeDtypeStruct((1, 1, 16), jnp.int32),
        jax.ShapeDtypeStruct((1, 1, 8), jnp.int32),
    ],
)


def kernel(points_xy):
    grids = _scatter_call(points_xy[:, 0], points_xy[:, 1])
    o0, o1, o2 = _reduce_call(grids, grids, grids, grids, grids, grids)
    return (o0.reshape(1, 32), o1.reshape(1, 16), o2.reshape(1, 8))


# speedup vs baseline: 1.2440x; 1.1634x over previous
import jax
import jax.numpy as jnp
import numpy as np
from jax import lax
from jax.experimental import pallas as pl
from jax.experimental.pallas import tpu as pltpu
from jax.experimental.pallas import tpu_sc as plsc

N_POINTS = 262144
NUM_CORES = 2
NUM_SUBCORES = 16
NUM_TILES = NUM_CORES * NUM_SUBCORES
PER_TILE = N_POINTS // NUM_TILES
LANES = 16
VEC_ITERS = PER_TILE // LANES

SIZES = (np.float32(0.1), np.float32(0.2), np.float32(0.4))
GRIDS = (1024, 512, 256)
BASES = (0, 1024 * 1024, 1024 * 1024 + 512 * 512)
C = 1024 * 1024 + 512 * 512 + 256 * 256
SHIFT = np.float32(51.2)

CHUNK = 128
N_ROWS = 3 * PER_TILE // CHUNK
ROWS_PER_RES = N_ROWS // 3
ROWS_PER_STREAM = 8
N_STREAMS = N_ROWS // ROWS_PER_STREAM
ZB = 2048
ZERO_PER_SUBCORE = C // NUM_SUBCORES
N_ZERO_COPIES = ZERO_PER_SUBCORE // ZB


def _scatter_body(xs_hbm, ys_hbm, out_hbm, xv, yv, idxb, ones, zb, grid_sh,
                  sem_ld, sem_sc, sem_z):
    cid = lax.axis_index("c")
    sid = lax.axis_index("s")
    wid = cid * NUM_SUBCORES + sid

    base = wid * PER_TILE
    d_x = pltpu.async_copy(xs_hbm.at[pl.ds(base, PER_TILE)], xv, sem_ld)
    d_y = pltpu.async_copy(ys_hbm.at[pl.ds(base, PER_TILE)], yv, sem_ld)

    @plsc.parallel_loop(0, ZB // LANES, unroll=8)
    def _fill_zb(i):
        zb[pl.ds(i * LANES, LANES)] = jnp.zeros((LANES,), jnp.float32)

    @plsc.parallel_loop(0, ROWS_PER_STREAM * CHUNK // LANES, unroll=8)
    def _fill_ones(i):
        ones[i // 8, pl.ds((i % 8) * LANES, LANES)] = jnp.ones(
            (LANES,), jnp.float32)

    ones_r = ones.at[0]

    zbase = sid * ZERO_PER_SUBCORE

    @plsc.parallel_loop(0, N_ZERO_COPIES, unroll=4)
    def _zero_fire(j):
        pltpu.async_copy(zb, grid_sh.at[pl.ds(zbase + j * ZB, ZB)], sem_z)

    d_x.wait()
    d_y.wait()

    def _quant(i):
        x = xv[pl.ds(i * LANES, LANES)]
        y = yv[pl.ds(i * LANES, LANES)]
        sx = x + SHIFT
        sy = y + SHIFT
        row = i // 8
        col = (i % 8) * LANES
        cx = (sx / SIZES[0]).astype(jnp.int32)
        cy = (sy / SIZES[0]).astype(jnp.int32)
        cx = jnp.minimum(jnp.maximum(cx, 0), GRIDS[0] - 1)
        cy = jnp.minimum(jnp.maximum(cy, 0), GRIDS[0] - 1)
        idxb[row, pl.ds(col, LANES)] = cy * 1024 + cx
        idxb[ROWS_PER_RES + row, pl.ds(col, LANES)] = (
            (cy >> 1) * 512 + (cx >> 1) + BASES[1])
        idxb[2 * ROWS_PER_RES + row, pl.ds(col, LANES)] = (
            (cy >> 2) * 256 + (cx >> 2) + BASES[2])

    with jax.named_scope("ph_quant"):
        plsc.parallel_loop(0, VEC_ITERS, unroll=4)(_quant)

    def _zero_drain(j, _):
        pltpu.make_async_copy(
            zb, grid_sh.at[pl.ds(zbase + j * ZB, ZB)], sem_z).wait()
        return 0

    with jax.named_scope("ph_zdrain"):
        lax.fori_loop(0, N_ZERO_COPIES, _zero_drain, 0)
        plsc.subcore_barrier()

    def _scatter_fire(j):
        pltpu.async_copy(ones_r, grid_sh.at[idxb.at[j]], sem_sc)

    with jax.named_scope("ph_sfire"):
        plsc.parallel_loop(0, N_ROWS, unroll=4)(_scatter_fire)

    def _scatter_drain(j, _):
        pltpu.make_async_copy(ones_r, grid_sh.at[idxb.at[j]], sem_sc).wait()
        return 0

    with jax.named_scope("ph_sdrain"):
        lax.fori_loop(0, N_ROWS, _scatter_drain, 0)
        plsc.subcore_barrier()

    with jax.named_scope("ph_wb"):
        pltpu.sync_copy(grid_sh.at[pl.ds(zbase, ZERO_PER_SUBCORE)],
                        out_hbm.at[pl.ds(cid * C + zbase, ZERO_PER_SUBCORE)])


_scatter_call = pl.kernel(
    _scatter_body,
    out_type=jax.ShapeDtypeStruct((NUM_CORES * C,), jnp.float32),
    mesh=plsc.VectorSubcoreMesh(core_axis_name="c", subcore_axis_name="s"),
    scratch_types=[
        pltpu.VMEM((PER_TILE,), jnp.float32),
        pltpu.VMEM((PER_TILE,), jnp.float32),
        pltpu.VMEM((N_ROWS, CHUNK), jnp.int32),
        pltpu.VMEM((ROWS_PER_STREAM, CHUNK), jnp.float32),
        pltpu.VMEM((ZB,), jnp.float32),
        pltpu.VMEM_SHARED((C,), jnp.float32),
        pltpu.SemaphoreType.DMA,
        pltpu.SemaphoreType.DMA,
        pltpu.SemaphoreType.DMA,
    ],
)

S0 = 32 * 1024
S1 = 32 * 512
S2 = 32 * 256


def _reduce_body(a0, a1, b0, b1, c0, c1, o0_ref, o1_ref, o2_ref):
    b = pl.program_id(0)

    def occ2(r0, r1, nrows):
        a = r0[...].reshape(nrows, 1024)
        b2 = r1[...].reshape(nrows, 1024)
        return ((a + b2) > 0.0).astype(jnp.float32)

    def put(ref, lane, val):
        li = lax.broadcasted_iota(jnp.int32, ref.shape, 2)
        ref[...] = jnp.where(li == lane, val.astype(jnp.int32), ref[...])

    oa = occ2(a0, a1, 64)
    put(o0_ref, 2 * b, jnp.sum(oa[:32]))
    put(o0_ref, 2 * b + 1, jnp.sum(oa[32:]))
    put(o1_ref, b, jnp.sum(occ2(b0, b1, 16)))
    put(o2_ref, b % 8, jnp.sum(occ2(c0, c1, 8)))


_reduce_call = pl.pallas_call(
    _reduce_body,
    grid=(16,),
    in_specs=[
        pl.BlockSpec((2 * S0,), lambda b: (b,)),
        pl.BlockSpec((2 * S0,), lambda b: (C // (2 * S0) + b,)),
        pl.BlockSpec((S1,), lambda b: (BASES[1] // S1 + b,)),
        pl.BlockSpec((S1,), lambda b: ((C + BASES[1]) // S1 + b,)),
        pl.BlockSpec((S2,), lambda b: (BASES[2] // S2 + b % 8,)),
        pl.BlockSpec((S2,), lambda b: ((C + BASES[2]) // S2 + b % 8,)),
    ],
    out_specs=[
        pl.BlockSpec((1, 1, 32), lambda b: (0, 0, 0)),
        pl.BlockSpec((1, 1, 16), lambda b: (0, 0, 0)),
        pl.BlockSpec((1, 1, 8), lambda b: (0, 0, 0)),
    ],
    out_shape=[
        jax.ShapeDtypeStruct((1, 1, 32), jnp.int32),
        jax.ShapeDtypeStruct((1, 1, 16), jnp.int32),
        jax.ShapeDtypeStruct((1, 1, 8), jnp.int32),
    ],
)


def kernel(points_xy):
    grids = _scatter_call(points_xy[:, 0], points_xy[:, 1])
    o0, o1, o2 = _reduce_call(grids, grids, grids, grids, grids, grids)
    return (o0.reshape(1, 32), o1.reshape(1, 16), o2.reshape(1, 8))


# speedup vs baseline: 1.2855x; 1.0334x over previous
import jax
import jax.numpy as jnp
import numpy as np
from jax import lax
from jax.experimental import pallas as pl
from jax.experimental.pallas import tpu as pltpu
from jax.experimental.pallas import tpu_sc as plsc

N_POINTS = 262144
NUM_CORES = 2
NUM_SUBCORES = 16
PER_TILE = N_POINTS // NUM_SUBCORES
N_PASSES = 2
PASS_PTS = PER_TILE // N_PASSES
LANES = 16
PASS_ITERS = PASS_PTS // LANES

SIZES = (np.float32(0.1), np.float32(0.2), np.float32(0.4))
GRIDS = (1024, 512, 256)
BASES = (0, 1024 * 1024, 1024 * 1024 + 512 * 512)
C = 1024 * 1024 + 512 * 512 + 256 * 256
C0 = BASES[1]
C1 = C - C0
SHIFT = np.float32(51.2)

CHUNK = 128
ROWS0 = PASS_PTS // CHUNK
ROWS1 = 2 * ROWS0
IPR = CHUNK // LANES
ZB = 2048
NZ0 = C0 // NUM_SUBCORES // ZB
NZ1 = C1 // NUM_SUBCORES // ZB
Z0 = C0 // NUM_SUBCORES
Z1 = C1 // NUM_SUBCORES


def _scatter_body(xs_hbm, ys_hbm, out_hbm, xv, yv, idxb, ones, zb, grid_sh,
                  sem_ld, sem_sc, sem_z):
    cid = lax.axis_index("c")
    sid = lax.axis_index("s")
    is0 = cid == 0
    tbase = sid * PER_TILE

    d_x = pltpu.async_copy(xs_hbm.at[pl.ds(tbase, PASS_PTS)], xv, sem_ld)
    d_y = pltpu.async_copy(ys_hbm.at[pl.ds(tbase, PASS_PTS)], yv, sem_ld)

    @plsc.parallel_loop(0, ZB // LANES, unroll=8)
    def _fill_zb(i):
        zb[pl.ds(i * LANES, LANES)] = jnp.zeros((LANES,), jnp.float32)

    @plsc.parallel_loop(0, CHUNK // LANES, unroll=8)
    def _fill_ones(i):
        ones[pl.ds(i * LANES, LANES)] = jnp.ones((LANES,), jnp.float32)

    zbase = jnp.where(is0, sid * Z0, C0 + sid * Z1)
    nz = jnp.where(is0, NZ0, NZ1)

    def _zero_fire(j, _):
        pltpu.async_copy(zb, grid_sh.at[pl.ds(zbase + j * ZB, ZB)], sem_z)
        return 0

    lax.fori_loop(0, nz, _zero_fire, 0)

    d_x.wait()
    d_y.wait()

    def _quant_pass(p):
        def _q(i):
            x = xv[pl.ds(i * LANES, LANES)]
            y = yv[pl.ds(i * LANES, LANES)]
            cx = ((x + SHIFT) / SIZES[0]).astype(jnp.int32)
            cy = ((y + SHIFT) / SIZES[0]).astype(jnp.int32)
            cx = jnp.minimum(jnp.maximum(cx, 0), GRIDS[0] - 1)
            cy = jnp.minimum(jnp.maximum(cy, 0), GRIDS[0] - 1)
            row = i // IPR
            col = (i % IPR) * LANES

            @pl.when(is0)
            def _():
                idxb[row, pl.ds(col, LANES)] = cy * 1024 + cx

            @pl.when(jnp.logical_not(is0))
            def _():
                idxb[row, pl.ds(col, LANES)] = (
                    (cy >> 1) * 512 + (cx >> 1) + BASES[1])
                idxb[ROWS0 + row, pl.ds(col, LANES)] = (
                    (cy >> 2) * 256 + (cx >> 2) + BASES[2])

        plsc.parallel_loop(0, PASS_ITERS, unroll=4)(_q)

    def _fire_pass(nrows):
        def _f(j, _):
            pltpu.async_copy(ones, grid_sh.at[idxb.at[j]], sem_sc)
            return 0

        lax.fori_loop(0, nrows, _f, 0)

    def _drain_pass(nrows):
        def _d(j, _):
            pltpu.make_async_copy(ones, grid_sh.at[idxb.at[j]],
                                  sem_sc).wait()
            return 0

        lax.fori_loop(0, nrows, _d, 0)

    nrows = jnp.where(is0, ROWS0, ROWS1)

    with jax.named_scope("ph_quant0"):
        _quant_pass(0)

    with jax.named_scope("ph_zdrain"):
        def _zero_drain(j, _):
            pltpu.make_async_copy(
                zb, grid_sh.at[pl.ds(zbase + j * ZB, ZB)], sem_z).wait()
            return 0

        lax.fori_loop(0, nz, _zero_drain, 0)
        plsc.subcore_barrier()

    d_x2 = pltpu.async_copy(xs_hbm.at[pl.ds(tbase + PASS_PTS, PASS_PTS)],
                            xv, sem_ld)
    d_y2 = pltpu.async_copy(ys_hbm.at[pl.ds(tbase + PASS_PTS, PASS_PTS)],
                            yv, sem_ld)
    with jax.named_scope("ph_sfire0"):
        _fire_pass(nrows)
    with jax.named_scope("ph_sdrain0"):
        _drain_pass(nrows)

    d_x2.wait()
    d_y2.wait()
    with jax.named_scope("ph_quant1"):
        _quant_pass(1)
    with jax.named_scope("ph_sfire1"):
        _fire_pass(nrows)
    with jax.named_scope("ph_sdrain1"):
        _drain_pass(nrows)
        plsc.subcore_barrier()

    with jax.named_scope("ph_wb"):
        zlen = jnp.where(is0, Z0, Z1)
        pltpu.sync_copy(grid_sh.at[pl.ds(zbase, zlen)],
                        out_hbm.at[pl.ds(zbase, zlen)])


_scatter_call = pl.kernel(
    _scatter_body,
    out_type=jax.ShapeDtypeStruct((C,), jnp.float32),
    mesh=plsc.VectorSubcoreMesh(core_axis_name="c", subcore_axis_name="s"),
    scratch_types=[
        pltpu.VMEM((PASS_PTS,), jnp.float32),
        pltpu.VMEM((PASS_PTS,), jnp.float32),
        pltpu.VMEM((ROWS1, CHUNK), jnp.int32),
        pltpu.VMEM((CHUNK,), jnp.float32),
        pltpu.VMEM((ZB,), jnp.float32),
        pltpu.VMEM_SHARED((C,), jnp.float32),
        pltpu.SemaphoreType.DMA,
        pltpu.SemaphoreType.DMA,
        pltpu.SemaphoreType.DMA,
    ],
)

S0 = 32 * 1024
S1 = 32 * 512
S2 = 32 * 256


def _reduce_body(a0, b0, c0, o0_ref, o1_ref, o2_ref):
    b = pl.program_id(0)

    def occ(r, nrows):
        return (r[...].reshape(nrows, 1024) > 0.0).astype(jnp.float32)

    def put(ref, lane, val):
        li = lax.broadcasted_iota(jnp.int32, ref.shape, 2)
        ref[...] = jnp.where(li == lane, val.astype(jnp.int32), ref[...])

    oa = occ(a0, 64)
    put(o0_ref, 2 * b, jnp.sum(oa[:32]))
    put(o0_ref, 2 * b + 1, jnp.sum(oa[32:]))
    put(o1_ref, b, jnp.sum(occ(b0, 16)))
    put(o2_ref, b % 8, jnp.sum(occ(c0, 8)))


_reduce_call = pl.pallas_call(
    _reduce_body,
    grid=(16,),
    in_specs=[
        pl.BlockSpec((2 * S0,), lambda b: (b,)),
        pl.BlockSpec((S1,), lambda b: (BASES[1] // S1 + b,)),
        pl.BlockSpec((S2,), lambda b: (BASES[2] // S2 + b % 8,)),
    ],
    out_specs=[
        pl.BlockSpec((1, 1, 32), lambda b: (0, 0, 0)),
        pl.BlockSpec((1, 1, 16), lambda b: (0, 0, 0)),
        pl.BlockSpec((1, 1, 8), lambda b: (0, 0, 0)),
    ],
    out_shape=[
        jax.ShapeDtypeStruct((1, 1, 32), jnp.int32),
        jax.ShapeDtypeStruct((1, 1, 16), jnp.int32),
        jax.ShapeDtypeStruct((1, 1, 8), jnp.int32),
    ],
)


def kernel(points_xy):
    grid = _scatter_call(points_xy[:, 0], points_xy[:, 1])
    o0, o1, o2 = _reduce_call(grid, grid, grid)
    return (o0.reshape(1, 32), o1.reshape(1, 16), o2.reshape(1, 8))


# speedup vs baseline: 1.4583x; 1.1344x over previous
import jax
import jax.numpy as jnp
import numpy as np
from jax import lax
from jax.experimental import pallas as pl
from jax.experimental.pallas import tpu as pltpu
from jax.experimental.pallas import tpu_sc as plsc

N_POINTS = 262144
NUM_CORES = 2
NUM_SUBCORES = 16
PER_TILE = N_POINTS // NUM_SUBCORES
N_PASSES = 2
PASS_PTS = PER_TILE // N_PASSES
LANES = 16
PASS_ITERS = PASS_PTS // LANES

SIZES = (np.float32(0.1), np.float32(0.2), np.float32(0.4))
GRIDS = (1024, 512, 256)
BASES = (0, 1024 * 1024, 1024 * 1024 + 512 * 512)
C = 1024 * 1024 + 512 * 512 + 256 * 256
C0 = BASES[1]
C1 = C - C0
SHIFT = np.float32(51.2)

CHUNK = 128
ROWS0 = PASS_PTS // CHUNK
ROWS1 = 2 * ROWS0
IPR = CHUNK // LANES
ZB = 2048
NZ0 = C0 // NUM_SUBCORES // ZB
NZ1 = C1 // NUM_SUBCORES // ZB
Z0 = C0 // NUM_SUBCORES
Z1 = C1 // NUM_SUBCORES


def _scatter_body(xs_hbm, ys_hbm, out_hbm, xv, yv, idxb, ones, zb, grid_sh,
                  sem_ld, sem_sc, sem_z):
    cid = lax.axis_index("c")
    sid = lax.axis_index("s")
    is0 = cid == 0
    tbase = sid * PER_TILE

    d_x = pltpu.async_copy(xs_hbm.at[pl.ds(tbase, PASS_PTS)], xv, sem_ld)
    d_y = pltpu.async_copy(ys_hbm.at[pl.ds(tbase, PASS_PTS)], yv, sem_ld)

    @plsc.parallel_loop(0, ZB // LANES, unroll=8)
    def _fill_zb(i):
        zb[pl.ds(i * LANES, LANES)] = jnp.zeros((LANES,), jnp.float32)

    @plsc.parallel_loop(0, CHUNK // LANES, unroll=8)
    def _fill_ones(i):
        ones[pl.ds(i * LANES, LANES)] = jnp.ones((LANES,), jnp.float32)

    zbase = jnp.where(is0, sid * Z0, C0 + sid * Z1)
    nz = jnp.where(is0, NZ0, NZ1)

    def _zero_fire(j, _):
        pltpu.async_copy(zb, grid_sh.at[pl.ds(zbase + j * ZB, ZB)], sem_z)
        return 0

    lax.fori_loop(0, nz, _zero_fire, 0)

    d_x.wait()
    d_y.wait()

    def _quant_pass(p):
        def _q(i):
            x = xv[pl.ds(i * LANES, LANES)]
            y = yv[pl.ds(i * LANES, LANES)]
            cx = ((x + SHIFT) / SIZES[0]).astype(jnp.int32)
            cy = ((y + SHIFT) / SIZES[0]).astype(jnp.int32)
            cx = jnp.minimum(jnp.maximum(cx, 0), GRIDS[0] - 1)
            cy = jnp.minimum(jnp.maximum(cy, 0), GRIDS[0] - 1)
            row = i // IPR
            col = (i % IPR) * LANES

            @pl.when(is0)
            def _():
                idxb[row, pl.ds(col, LANES)] = cy * 1024 + cx

            @pl.when(jnp.logical_not(is0))
            def _():
                idxb[row, pl.ds(col, LANES)] = (
                    (cy >> 1) * 512 + (cx >> 1) + BASES[1])
                idxb[ROWS0 + row, pl.ds(col, LANES)] = (
                    (cy >> 2) * 256 + (cx >> 2) + BASES[2])

        plsc.parallel_loop(0, PASS_ITERS, unroll=4)(_q)

    def _fire_pass(nrows):
        def _f(j, _):
            pltpu.async_copy(ones, grid_sh.at[idxb.at[j]], sem_sc)
            return 0

        lax.fori_loop(0, nrows, _f, 0)

    def _drain_pass(nrows):
        def _d(j, _):
            pltpu.make_async_copy(ones, grid_sh.at[idxb.at[j]],
                                  sem_sc).wait()
            return 0

        lax.fori_loop(0, nrows, _d, 0)

    nrows = jnp.where(is0, ROWS0, ROWS1)

    with jax.named_scope("ph_quant0"):
        _quant_pass(0)

    with jax.named_scope("ph_zdrain"):
        def _zero_drain(j, _):
            pltpu.make_async_copy(
                zb, grid_sh.at[pl.ds(zbase + j * ZB, ZB)], sem_z).wait()
            return 0

        lax.fori_loop(0, nz, _zero_drain, 0)
        plsc.subcore_barrier()

    d_x2 = pltpu.async_copy(xs_hbm.at[pl.ds(tbase + PASS_PTS, PASS_PTS)],
                            xv, sem_ld)
    d_y2 = pltpu.async_copy(ys_hbm.at[pl.ds(tbase + PASS_PTS, PASS_PTS)],
                            yv, sem_ld)
    with jax.named_scope("ph_sfire0"):
        _fire_pass(nrows)
    with jax.named_scope("ph_sdrain0"):
        _drain_pass(nrows)

    d_x2.wait()
    d_y2.wait()
    with jax.named_scope("ph_quant1"):
        _quant_pass(1)
    with jax.named_scope("ph_sfire1"):
        _fire_pass(nrows)
    with jax.named_scope("ph_sdrain1"):
        _drain_pass(nrows)
        plsc.subcore_barrier()

    with jax.named_scope("ph_wb"):
        zlen = jnp.where(is0, Z0, Z1)
        pltpu.sync_copy(grid_sh.at[pl.ds(zbase, zlen)],
                        out_hbm.at[pl.ds(zbase, zlen)])


_scatter_call = pl.kernel(
    _scatter_body,
    out_type=jax.ShapeDtypeStruct((C,), jnp.float32),
    mesh=plsc.VectorSubcoreMesh(core_axis_name="c", subcore_axis_name="s"),
    scratch_types=[
        pltpu.VMEM((PASS_PTS,), jnp.float32),
        pltpu.VMEM((PASS_PTS,), jnp.float32),
        pltpu.VMEM((ROWS1, CHUNK), jnp.int32),
        pltpu.VMEM((CHUNK,), jnp.float32),
        pltpu.VMEM((ZB,), jnp.float32),
        pltpu.VMEM_SHARED((C,), jnp.float32),
        pltpu.SemaphoreType.DMA,
        pltpu.SemaphoreType.DMA,
        pltpu.SemaphoreType.DMA,
    ],
)

S0 = 32 * 1024
S1 = 32 * 512
S2 = 32 * 256


def _reduce_body(a0, b0, c0, o0_ref, o1_ref, o2_ref):
    b = pl.program_id(0)

    def occ(r, nrows):
        return (r[...].reshape(nrows, 1024) > 0.0).astype(jnp.float32)

    def put(ref, lane, val):
        li = lax.broadcasted_iota(jnp.int32, ref.shape, 2)
        ref[...] = jnp.where(li == lane, val.astype(jnp.int32), ref[...])

    oa = occ(a0, 256)
    for k in range(8):
        put(o0_ref, 8 * b + k, jnp.sum(oa[32 * k:32 * (k + 1)]))
    ob = occ(b0, 64)
    for k in range(4):
        put(o1_ref, 4 * b + k, jnp.sum(ob[16 * k:16 * (k + 1)]))
    oc = occ(c0, 16)
    for k in range(2):
        put(o2_ref, 2 * b + k, jnp.sum(oc[8 * k:8 * (k + 1)]))


_reduce_call = pl.pallas_call(
    _reduce_body,
    grid=(4,),
    in_specs=[
        pl.BlockSpec((8 * S0,), lambda b: (b,)),
        pl.BlockSpec((4 * S1,), lambda b: (BASES[1] // (4 * S1) + b,)),
        pl.BlockSpec((2 * S2,), lambda b: (BASES[2] // (2 * S2) + b,)),
    ],
    out_specs=[
        pl.BlockSpec((1, 1, 32), lambda b: (0, 0, 0)),
        pl.BlockSpec((1, 1, 16), lambda b: (0, 0, 0)),
        pl.BlockSpec((1, 1, 8), lambda b: (0, 0, 0)),
    ],
    out_shape=[
        jax.ShapeDtypeStruct((1, 1, 32), jnp.int32),
        jax.ShapeDtypeStruct((1, 1, 16), jnp.int32),
        jax.ShapeDtypeStruct((1, 1, 8), jnp.int32),
    ],
)


def kernel(points_xy):
    grid = _scatter_call(points_xy[:, 0], points_xy[:, 1])
    o0, o1, o2 = _reduce_call(grid, grid, grid)
    return (o0.reshape(1, 32), o1.reshape(1, 16), o2.reshape(1, 8))


# speedup vs baseline: 1.4590x; 1.0005x over previous
import jax
import jax.numpy as jnp
import numpy as np
from jax import lax
from jax.experimental import pallas as pl
from jax.experimental.pallas import tpu as pltpu
from jax.experimental.pallas import tpu_sc as plsc

N_POINTS = 262144
NUM_CORES = 2
NUM_SUBCORES = 16
PER_TILE = N_POINTS // NUM_SUBCORES
N_PASSES = 2
PASS_PTS = PER_TILE // N_PASSES
LANES = 16
PASS_ITERS = PASS_PTS // LANES

SIZES = (np.float32(0.1), np.float32(0.2), np.float32(0.4))
GRIDS = (1024, 512, 256)
BASES = (0, 1024 * 1024, 1024 * 1024 + 512 * 512)
C = 1024 * 1024 + 512 * 512 + 256 * 256
C0 = BASES[1]
C1 = C - C0
SHIFT = np.float32(51.2)
INV0 = np.float32(1.0) / SIZES[0]

CHUNK = 128
ROWS0 = PASS_PTS // CHUNK
ROWS1 = 2 * ROWS0
IPR = CHUNK // LANES
ZB = 2048
NZ0 = C0 // NUM_SUBCORES // ZB
NZ1 = C1 // NUM_SUBCORES // ZB
Z0 = C0 // NUM_SUBCORES
Z1 = C1 // NUM_SUBCORES


def _scatter_body(xs_hbm, ys_hbm, out_hbm, xv, yv, idxb, ones, zb, grid_sh,
                  sem_ld, sem_sc, sem_z):
    cid = lax.axis_index("c")
    sid = lax.axis_index("s")
    is0 = cid == 0
    tbase = sid * PER_TILE

    d_x = pltpu.async_copy(xs_hbm.at[pl.ds(tbase, PASS_PTS)], xv, sem_ld)
    d_y = pltpu.async_copy(ys_hbm.at[pl.ds(tbase, PASS_PTS)], yv, sem_ld)

    @plsc.parallel_loop(0, ZB // LANES, unroll=8)
    def _fill_zb(i):
        zb[pl.ds(i * LANES, LANES)] = jnp.zeros((LANES,), jnp.float32)

    @plsc.parallel_loop(0, CHUNK // LANES, unroll=8)
    def _fill_ones(i):
        ones[pl.ds(i * LANES, LANES)] = jnp.ones((LANES,), jnp.float32)

    zbase = jnp.where(is0, sid * Z0, C0 + sid * Z1)
    nz = jnp.where(is0, NZ0, NZ1)

    def _zero_fire(j, _):
        pltpu.async_copy(zb, grid_sh.at[pl.ds(zbase + j * ZB, ZB)], sem_z)
        return 0

    lax.fori_loop(0, nz, _zero_fire, 0)

    d_x.wait()
    d_y.wait()

    def _quant_pass(p):
        def _q(i):
            x = xv[pl.ds(i * LANES, LANES)]
            y = yv[pl.ds(i * LANES, LANES)]
            cx = ((x + SHIFT) * INV0).astype(jnp.int32)
            cy = ((y + SHIFT) * INV0).astype(jnp.int32)
            cx = jnp.minimum(jnp.maximum(cx, 0), GRIDS[0] - 1)
            cy = jnp.minimum(jnp.maximum(cy, 0), GRIDS[0] - 1)
            row = i // IPR
            col = (i % IPR) * LANES

            @pl.when(is0)
            def _():
                idxb[row, pl.ds(col, LANES)] = cy * 1024 + cx

            @pl.when(jnp.logical_not(is0))
            def _():
                idxb[row, pl.ds(col, LANES)] = (
                    (cy >> 1) * 512 + (cx >> 1) + BASES[1])
                idxb[ROWS0 + row, pl.ds(col, LANES)] = (
                    (cy >> 2) * 256 + (cx >> 2) + BASES[2])

        plsc.parallel_loop(0, PASS_ITERS, unroll=4)(_q)

    def _fire_pass(nrows):
        def _f(j, _):
            pltpu.async_copy(ones, grid_sh.at[idxb.at[j]], sem_sc)
            return 0

        lax.fori_loop(0, nrows, _f, 0)

    def _drain_pass(nrows):
        def _d(j, _):
            pltpu.make_async_copy(ones, grid_sh.at[idxb.at[j]],
                                  sem_sc).wait()
            return 0

        lax.fori_loop(0, nrows, _d, 0)

    nrows = jnp.where(is0, ROWS0, ROWS1)

    with jax.named_scope("ph_quant0"):
        _quant_pass(0)

    with jax.named_scope("ph_zdrain"):
        def _zero_drain(j, _):
            pltpu.make_async_copy(
                zb, grid_sh.at[pl.ds(zbase + j * ZB, ZB)], sem_z).wait()
            return 0

        lax.fori_loop(0, nz, _zero_drain, 0)
        plsc.subcore_barrier()

    d_x2 = pltpu.async_copy(xs_hbm.at[pl.ds(tbase + PASS_PTS, PASS_PTS)],
                            xv, sem_ld)
    d_y2 = pltpu.async_copy(ys_hbm.at[pl.ds(tbase + PASS_PTS, PASS_PTS)],
                            yv, sem_ld)
    with jax.named_scope("ph_sfire0"):
        _fire_pass(nrows)
    with jax.named_scope("ph_sdrain0"):
        _drain_pass(nrows)

    d_x2.wait()
    d_y2.wait()
    with jax.named_scope("ph_quant1"):
        _quant_pass(1)
    with jax.named_scope("ph_sfire1"):
        _fire_pass(nrows)
    with jax.named_scope("ph_sdrain1"):
        _drain_pass(nrows)
        plsc.subcore_barrier()

    with jax.named_scope("ph_wb"):
        zlen = jnp.where(is0, Z0, Z1)
        pltpu.sync_copy(grid_sh.at[pl.ds(zbase, zlen)],
                        out_hbm.at[pl.ds(zbase, zlen)])


_scatter_call = pl.kernel(
    _scatter_body,
    out_type=jax.ShapeDtypeStruct((C,), jnp.float32),
    mesh=plsc.VectorSubcoreMesh(core_axis_name="c", subcore_axis_name="s"),
    scratch_types=[
        pltpu.VMEM((PASS_PTS,), jnp.float32),
        pltpu.VMEM((PASS_PTS,), jnp.float32),
        pltpu.VMEM((ROWS1, CHUNK), jnp.int32),
        pltpu.VMEM((CHUNK,), jnp.float32),
        pltpu.VMEM((ZB,), jnp.float32),
        pltpu.VMEM_SHARED((C,), jnp.float32),
        pltpu.SemaphoreType.DMA,
        pltpu.SemaphoreType.DMA,
        pltpu.SemaphoreType.DMA,
    ],
)

S0 = 32 * 1024
S1 = 32 * 512
S2 = 32 * 256


def _reduce_body(a0, b0, c0, o0_ref, o1_ref, o2_ref):
    b = pl.program_id(0)

    def occ(r, nrows):
        return (r[...].reshape(nrows, 1024) > 0.0).astype(jnp.float32)

    def put(ref, lane, val):
        li = lax.broadcasted_iota(jnp.int32, ref.shape, 2)
        ref[...] = jnp.where(li == lane, val.astype(jnp.int32), ref[...])

    oa = occ(a0, 256)
    for k in range(8):
        put(o0_ref, 8 * b + k, jnp.sum(oa[32 * k:32 * (k + 1)]))
    ob = occ(b0, 64)
    for k in range(4):
        put(o1_ref, 4 * b + k, jnp.sum(ob[16 * k:16 * (k + 1)]))
    oc = occ(c0, 16)
    for k in range(2):
        put(o2_ref, 2 * b + k, jnp.sum(oc[8 * k:8 * (k + 1)]))


_reduce_call = pl.pallas_call(
    _reduce_body,
    grid=(4,),
    in_specs=[
        pl.BlockSpec((8 * S0,), lambda b: (b,)),
        pl.BlockSpec((4 * S1,), lambda b: (BASES[1] // (4 * S1) + b,)),
        pl.BlockSpec((2 * S2,), lambda b: (BASES[2] // (2 * S2) + b,)),
    ],
    out_specs=[
        pl.BlockSpec((1, 1, 32), lambda b: (0, 0, 0)),
        pl.BlockSpec((1, 1, 16), lambda b: (0, 0, 0)),
        pl.BlockSpec((1, 1, 8), lambda b: (0, 0, 0)),
    ],
    out_shape=[
        jax.ShapeDtypeStruct((1, 1, 32), jnp.int32),
        jax.ShapeDtypeStruct((1, 1, 16), jnp.int32),
        jax.ShapeDtypeStruct((1, 1, 8), jnp.int32),
    ],
)


def kernel(points_xy):
    grid = _scatter_call(points_xy[:, 0], points_xy[:, 1])
    o0, o1, o2 = _reduce_call(grid, grid, grid)
    return (o0.reshape(1, 32), o1.reshape(1, 16), o2.reshape(1, 8))


# speedup vs baseline: 1.4910x; 1.0220x over previous
import jax
import jax.numpy as jnp
import numpy as np
from jax import lax
from jax.experimental import pallas as pl
from jax.experimental.pallas import tpu as pltpu
from jax.experimental.pallas import tpu_sc as plsc

N_POINTS = 262144
NUM_CORES = 2
NUM_SUBCORES = 16
PER_TILE = N_POINTS // NUM_SUBCORES
N_PASSES = 4
PASS_PTS = PER_TILE // N_PASSES
LANES = 16
PASS_ITERS = PASS_PTS // LANES

SIZES = (np.float32(0.1), np.float32(0.2), np.float32(0.4))
GRIDS = (1024, 512, 256)
BASES = (0, 1024 * 1024, 1024 * 1024 + 512 * 512)
C = 1024 * 1024 + 512 * 512 + 256 * 256
C0 = BASES[1]
C1 = C - C0
SHIFT = np.float32(51.2)
INV0 = np.float32(1.0) / SIZES[0]

CHUNK = 128
ROWS0 = PASS_PTS // CHUNK
ROWS1 = 2 * ROWS0
IPR = CHUNK // LANES
ZB = 2048
NZ0 = C0 // NUM_SUBCORES // ZB
NZ1 = C1 // NUM_SUBCORES // ZB
Z0 = C0 // NUM_SUBCORES
Z1 = C1 // NUM_SUBCORES


def _scatter_body(xs_hbm, ys_hbm, out_hbm, xv, yv, idxb, ones, zb, grid_sh,
                  sem_ld, sem_a, sem_b, sem_z):
    cid = lax.axis_index("c")
    sid = lax.axis_index("s")
    is0 = cid == 0
    tbase = sid * PER_TILE
    sems = (sem_a, sem_b)

    def load_pass(pp):
        k = pp % 2
        dx = pltpu.async_copy(
            xs_hbm.at[pl.ds(tbase + pp * PASS_PTS, PASS_PTS)], xv.at[k],
            sem_ld)
        dy = pltpu.async_copy(
            ys_hbm.at[pl.ds(tbase + pp * PASS_PTS, PASS_PTS)], yv.at[k],
            sem_ld)
        return dx, dy

    d0 = load_pass(0)

    @plsc.parallel_loop(0, ZB // LANES, unroll=8)
    def _fill_zb(i):
        zb[pl.ds(i * LANES, LANES)] = jnp.zeros((LANES,), jnp.float32)

    @plsc.parallel_loop(0, CHUNK // LANES, unroll=8)
    def _fill_ones(i):
        ones[pl.ds(i * LANES, LANES)] = jnp.ones((LANES,), jnp.float32)

    zbase = jnp.where(is0, sid * Z0, C0 + sid * Z1)
    nz = jnp.where(is0, NZ0, NZ1)

    def _zero_fire(j, _):
        pltpu.async_copy(zb, grid_sh.at[pl.ds(zbase + j * ZB, ZB)], sem_z)
        return 0

    lax.fori_loop(0, nz, _zero_fire, 0)

    nrows = jnp.where(is0, ROWS0, ROWS1)

    def quant_pass(pp):
        k = pp % 2

        def _q(i):
            x = xv[k, pl.ds(i * LANES, LANES)]
            y = yv[k, pl.ds(i * LANES, LANES)]
            cx = ((x + SHIFT) / SIZES[0]).astype(jnp.int32)
            cy = ((y + SHIFT) / SIZES[0]).astype(jnp.int32)
            cx = jnp.minimum(jnp.maximum(cx, 0), GRIDS[0] - 1)
            cy = jnp.minimum(jnp.maximum(cy, 0), GRIDS[0] - 1)
            row = i // IPR
            col = (i % IPR) * LANES

            @pl.when(is0)
            def _():
                idxb[k, row, pl.ds(col, LANES)] = cy * 1024 + cx

            @pl.when(jnp.logical_not(is0))
            def _():
                idxb[k, row, pl.ds(col, LANES)] = (
                    (cy >> 1) * 512 + (cx >> 1) + BASES[1])
                idxb[k, ROWS0 + row, pl.ds(col, LANES)] = (
                    (cy >> 2) * 256 + (cx >> 2) + BASES[2])

        plsc.parallel_loop(0, PASS_ITERS, unroll=4)(_q)

    def fire_pass(pp):
        k = pp % 2

        def _f(j, _):
            pltpu.async_copy(ones, grid_sh.at[idxb.at[k, j]], sems[k])
            return 0

        lax.fori_loop(0, nrows, _f, 0)

    def drain_pass(pp):
        k = pp % 2

        def _d(j, _):
            pltpu.make_async_copy(ones, grid_sh.at[idxb.at[k, j]],
                                  sems[k]).wait()
            return 0

        lax.fori_loop(0, nrows, _d, 0)

    d0[0].wait()
    d0[1].wait()
    d1 = load_pass(1)
    with jax.named_scope("ph_quant0"):
        quant_pass(0)

    with jax.named_scope("ph_zdrain"):
        def _zero_drain(j, _):
            pltpu.make_async_copy(
                zb, grid_sh.at[pl.ds(zbase + j * ZB, ZB)], sem_z).wait()
            return 0

        lax.fori_loop(0, nz, _zero_drain, 0)
        plsc.subcore_barrier()

    fire_pass(0)
    d1[0].wait()
    d1[1].wait()
    d2 = load_pass(2)
    with jax.named_scope("ph_quant1"):
        quant_pass(1)
    fire_pass(1)
    drain_pass(0)
    d2[0].wait()
    d2[1].wait()
    d3 = load_pass(3)
    with jax.named_scope("ph_quant2"):
        quant_pass(2)
    fire_pass(2)
    drain_pass(1)
    d3[0].wait()
    d3[1].wait()
    with jax.named_scope("ph_quant3"):
        quant_pass(3)
    fire_pass(3)
    with jax.named_scope("ph_sdrain"):
        drain_pass(2)
        drain_pass(3)
        plsc.subcore_barrier()

    with jax.named_scope("ph_wb"):
        zlen = jnp.where(is0, Z0, Z1)
        pltpu.sync_copy(grid_sh.at[pl.ds(zbase, zlen)],
                        out_hbm.at[pl.ds(zbase, zlen)])


_scatter_call = pl.kernel(
    _scatter_body,
    out_type=jax.ShapeDtypeStruct((C,), jnp.float32),
    mesh=plsc.VectorSubcoreMesh(core_axis_name="c", subcore_axis_name="s"),
    scratch_types=[
        pltpu.VMEM((2, PASS_PTS), jnp.float32),
        pltpu.VMEM((2, PASS_PTS), jnp.float32),
        pltpu.VMEM((2, ROWS1, CHUNK), jnp.int32),
        pltpu.VMEM((CHUNK,), jnp.float32),
        pltpu.VMEM((ZB,), jnp.float32),
        pltpu.VMEM_SHARED((C,), jnp.float32),
        pltpu.SemaphoreType.DMA,
        pltpu.SemaphoreType.DMA,
        pltpu.SemaphoreType.DMA,
        pltpu.SemaphoreType.DMA,
    ],
)

S0 = 32 * 1024
S1 = 32 * 512
S2 = 32 * 256


def _reduce_body(a0, b0, c0, o0_ref, o1_ref, o2_ref):
    b = pl.program_id(0)

    def occ(r, nrows):
        return (r[...].reshape(nrows, 1024) > 0.0).astype(jnp.float32)

    def put(ref, lane, val):
        li = lax.broadcasted_iota(jnp.int32, ref.shape, 2)
        ref[...] = jnp.where(li == lane, val.astype(jnp.int32), ref[...])

    oa = occ(a0, 256)
    for k in range(8):
        put(o0_ref, 8 * b + k, jnp.sum(oa[32 * k:32 * (k + 1)]))
    ob = occ(b0, 64)
    for k in range(4):
        put(o1_ref, 4 * b + k, jnp.sum(ob[16 * k:16 * (k + 1)]))
    oc = occ(c0, 16)
    for k in range(2):
        put(o2_ref, 2 * b + k, jnp.sum(oc[8 * k:8 * (k + 1)]))


_reduce_call = pl.pallas_call(
    _reduce_body,
    grid=(4,),
    in_specs=[
        pl.BlockSpec((8 * S0,), lambda b: (b,)),
        pl.BlockSpec((4 * S1,), lambda b: (BASES[1] // (4 * S1) + b,)),
        pl.BlockSpec((2 * S2,), lambda b: (BASES[2] // (2 * S2) + b,)),
    ],
    out_specs=[
        pl.BlockSpec((1, 1, 32), lambda b: (0, 0, 0)),
        pl.BlockSpec((1, 1, 16), lambda b: (0, 0, 0)),
        pl.BlockSpec((1, 1, 8), lambda b: (0, 0, 0)),
    ],
    out_shape=[
        jax.ShapeDtypeStruct((1, 1, 32), jnp.int32),
        jax.ShapeDtypeStruct((1, 1, 16), jnp.int32),
        jax.ShapeDtypeStruct((1, 1, 8), jnp.int32),
    ],
)


def kernel(points_xy):
    grid = _scatter_call(points_xy[:, 0], points_xy[:, 1])
    o0, o1, o2 = _reduce_call(grid, grid, grid)
    return (o0.reshape(1, 32), o1.reshape(1, 16), o2.reshape(1, 8))
